# baseline (speedup 1.0000x reference)
import functools

import jax
import jax.numpy as jnp
from jax import lax
from jax.experimental import pallas as pl
from jax.experimental.pallas import tpu as pltpu
from jax.experimental.pallas import tpu_sc as plsc

_B = 4
_N = 8192
_C = 32
_S = 512
_K = 32
_R = 0.2
_CP = 48
_ST = 128
_RT = 4096
_NROW = _B * _S * _K


def _fps_body(xyzt_ref, far0_ref, newx_ref, dist_ref):
    x = xyzt_ref[:, 0, :]
    y = xyzt_ref[:, 1, :]
    z = xyzt_ref[:, 2, :]
    n_iota = lax.broadcasted_iota(jnp.int32, (_B, _N), 1)
    s_iota = lax.broadcasted_iota(jnp.int32, (_B, _S), 1)
    far0 = jnp.min(
        jnp.where(far0_ref[...] > 0.0, n_iota, jnp.int32(_N)), axis=1,
        keepdims=True)
    dist_ref[...] = jnp.full((_B, _N), 1e10, jnp.float32)

    def body(i, st):
        far, nx, ny, nz = st
        msk = n_iota == far
        cx = jnp.sum(jnp.where(msk, x, 0.0), axis=1, keepdims=True)
        cy = jnp.sum(jnp.where(msk, y, 0.0), axis=1, keepdims=True)
        cz = jnp.sum(jnp.where(msk, z, 0.0), axis=1, keepdims=True)
        s_msk = s_iota == i
        nx = jnp.where(s_msk, cx, nx)
        ny = jnp.where(s_msk, cy, ny)
        nz = jnp.where(s_msk, cz, nz)
        dx = x - cx
        dy = y - cy
        dz = z - cz
        d = dx * dx + dy * dy + dz * dz
        dist = jnp.minimum(dist_ref[...], d)
        dist_ref[...] = dist
        m = jnp.max(dist, axis=1, keepdims=True)
        far = jnp.min(jnp.where(dist == m, n_iota, jnp.int32(_N)), axis=1,
                      keepdims=True)
        return (far, nx, ny, nz)

    zero = jnp.zeros((_B, _S), jnp.float32)
    _, nx, ny, nz = lax.fori_loop(0, _S, body, (far0, zero, zero, zero))
    newx_ref[:, 0, :] = nx
    newx_ref[:, 1, :] = ny
    newx_ref[:, 2, :] = nz


def _run_fps(xyzt, far0_oh):
    return pl.pallas_call(
        _fps_body,
        out_shape=jax.ShapeDtypeStruct((_B, 3, _S), jnp.float32),
        scratch_shapes=[pltpu.VMEM((_B, _N), jnp.float32)],
    )(xyzt, far0_oh)


def _bf(v):
    return v.astype(jnp.bfloat16).astype(jnp.float32)


def _two_sum(p, q):
    s = p + q
    pp = s - q
    qq = s - pp
    return s, (p - pp) + (q - qq)


def _ball_body(xyzt_ref, nc_ref, idx_ref, dist_ref):
    b = pl.program_id(0)
    x = xyzt_ref[0, 0:1, :]
    y = xyzt_ref[0, 1:2, :]
    z = xyzt_ref[0, 2:3, :]
    c = nc_ref[0]
    l3 = lax.broadcasted_iota(jnp.int32, (_ST, 3), 1)
    cx = jnp.sum(jnp.where(l3 == 0, c, 0.0), axis=1, keepdims=True)
    cy = jnp.sum(jnp.where(l3 == 1, c, 0.0), axis=1, keepdims=True)
    cz = jnp.sum(jnp.where(l3 == 2, c, 0.0), axis=1, keepdims=True)
    a2 = (cx * cx + cy * cy) + cz * cz
    b2 = (x * x + y * y) + z * z
    p0 = _bf(cx) * _bf(x)
    p1 = _bf(cy) * _bf(y)
    p2 = _bf(cz) * _bf(z)
    s01, e01 = _two_sum(p0, p1)
    t, e2 = _two_sum(s01, p2)
    ab = t + (e01 + e2)
    dist_ref[...] = jnp.maximum((a2 + b2) - 2.0 * ab, 0.0)

    n_iota = lax.broadcasted_iota(jnp.int32, (_ST, _N), 1) + b * _N
    k_iota = lax.broadcasted_iota(jnp.int32, (_ST, _K), 1)
    big = jnp.int32(1 << 30)

    def rnd(k, st):
        out, first, dprev, iprev = st
        d = dist_ref[...]
        live = (d > dprev) | ((d == dprev) & (n_iota > iprev))
        m = jnp.min(jnp.where(live, d, jnp.float32(3e38)), axis=1,
                    keepdims=True)
        sel = jnp.min(jnp.where(live & (d == m), n_iota, big), axis=1,
                      keepdims=True)
        first = jnp.where(k == 0, sel, first)
        chosen = jnp.where(jnp.sqrt(m) > _R, first, sel)
        out = jnp.where(k_iota == k, chosen, out)
        return (out, first, m, sel)

    out0 = jnp.zeros((_ST, _K), jnp.int32)
    f0 = jnp.zeros((_ST, 1), jnp.int32)
    dp0 = jnp.full((_ST, 1), -1.0, jnp.float32)
    ip0 = jnp.full((_ST, 1), -1, jnp.int32)
    out, _, _, _ = lax.fori_loop(0, _K, rnd, (out0, f0, dp0, ip0))
    idx_ref[0] = out


def _run_ball(xyzt, nc):
    grid = (_B, _S // _ST)
    return pl.pallas_call(
        _ball_body,
        grid=grid,
        in_specs=[
            pl.BlockSpec((1, 3, _N), lambda b, t: (b, 0, 0)),
            pl.BlockSpec((1, _ST, 3), lambda b, t: (b, t, 0)),
        ],
        out_specs=pl.BlockSpec((1, _ST, _K), lambda b, t: (b, t, 0)),
        out_shape=jax.ShapeDtypeStruct((_B, _S, _K), jnp.int32),
        scratch_shapes=[pltpu.VMEM((_ST, _N), jnp.float32)],
    )(xyzt, nc)


_NW = 32
_CH = 128
_NCHUNK = _NROW // (_NW * _CH)


def _sc_gather_body(pts_hbm, gidx_hbm, out_hbm, idx_v, rows_v, sem):
    wid = lax.axis_index("s") * 2 + lax.axis_index("c")

    def body(j, carry):
        base = (wid * _NCHUNK + j) * _CH
        pltpu.sync_copy(gidx_hbm.at[pl.ds(base, _CH)], idx_v)
        pltpu.async_copy(pts_hbm.at[idx_v], rows_v, sem).wait()
        pltpu.sync_copy(rows_v, out_hbm.at[pl.ds(base, _CH)])
        return carry

    lax.fori_loop(0, _NCHUNK, body, 0)


def _run_sc_gather(pts, gidx):
    mesh = plsc.VectorSubcoreMesh(core_axis_name="c", subcore_axis_name="s")
    fn = functools.partial(
        pl.kernel,
        mesh=mesh,
        compiler_params=pltpu.CompilerParams(use_tc_tiling_on_sc=False),
        out_type=jax.ShapeDtypeStruct((_NROW, _CP), jnp.float32),
        scratch_types=[
            pltpu.VMEM((_CH,), jnp.int32),
            pltpu.VMEM((_CH, _CP), jnp.float32),
            pltpu.SemaphoreType.DMA,
        ],
    )(_sc_gather_body)
    return fn(pts, gidx)


def _mlpA_body(g_ref, ctr_ref, w_ref, b_ref, y_ref, st_ref, acc_ref):
    t = pl.program_id(0)

    @pl.when(t == 0)
    def _():
        acc_ref[...] = jnp.zeros((8, 128), jnp.float32)

    xv = g_ref[...] - ctr_ref[...]
    y = lax.dot_general(xv, w_ref[...], (((1,), (0,)), ((), ())),
                        preferred_element_type=jnp.float32,
                        precision=lax.Precision.HIGHEST) + b_ref[...]
    y_ref[...] = y
    acc_ref[0:1, 0:_C] += jnp.sum(y, axis=0, keepdims=True)
    acc_ref[1:2, 0:_C] += jnp.sum(y * y, axis=0, keepdims=True)

    @pl.when(t == pl.num_programs(0) - 1)
    def _():
        st_ref[...] = acc_ref[...]


def _run_mlpA(g, ctr, w1p, b1r):
    grid = (_NROW // _RT,)
    return pl.pallas_call(
        _mlpA_body,
        grid=grid,
        in_specs=[
            pl.BlockSpec((_RT, _CP), lambda t: (t, 0)),
            pl.BlockSpec((_RT, _CP), lambda t: (t, 0)),
            pl.BlockSpec((_CP, _C), lambda t: (0, 0)),
            pl.BlockSpec((1, _C), lambda t: (0, 0)),
        ],
        out_specs=[
            pl.BlockSpec((_RT, _C), lambda t: (t, 0)),
            pl.BlockSpec((8, 128), lambda t: (0, 0)),
        ],
        out_shape=[
            jax.ShapeDtypeStruct((_NROW, _C), jnp.float32),
            jax.ShapeDtypeStruct((8, 128), jnp.float32),
        ],
        scratch_shapes=[pltpu.VMEM((8, 128), jnp.float32)],
    )(g, ctr, w1p, b1r)


def _mlpB_body(y_ref, st_ref, g_ref, be_ref, w_ref, b_ref, z_ref, st2_ref,
               acc_ref):
    t = pl.program_id(0)

    @pl.when(t == 0)
    def _():
        acc_ref[...] = jnp.zeros((8, 128), jnp.float32)

    n = jnp.float32(_NROW)
    mean = st_ref[0:1, 0:_C] / n
    var = st_ref[1:2, 0:_C] / n - mean * mean
    xn = (y_ref[...] - mean) / jnp.sqrt(var + 1e-5) * g_ref[...] + be_ref[...]
    xr = jnp.maximum(xn, 0.0)
    z = lax.dot_general(xr, w_ref[...], (((1,), (0,)), ((), ())),
                        preferred_element_type=jnp.float32,
                        precision=lax.Precision.HIGHEST) + b_ref[...]
    z_ref[...] = z
    acc_ref[0:1, 0:2 * _C] += jnp.sum(z, axis=0, keepdims=True)
    acc_ref[1:2, 0:2 * _C] += jnp.sum(z * z, axis=0, keepdims=True)

    @pl.when(t == pl.num_programs(0) - 1)
    def _():
        st2_ref[...] = acc_ref[...]


def _run_mlpB(y1, st1, g1r, be1r, w2t, b2r):
    grid = (_NROW // _RT,)
    return pl.pallas_call(
        _mlpB_body,
        grid=grid,
        in_specs=[
            pl.BlockSpec((_RT, _C), lambda t: (t, 0)),
            pl.BlockSpec((8, 128), lambda t: (0, 0)),
            pl.BlockSpec((1, _C), lambda t: (0, 0)),
            pl.BlockSpec((1, _C), lambda t: (0, 0)),
            pl.BlockSpec((_C, 2 * _C), lambda t: (0, 0)),
            pl.BlockSpec((1, 2 * _C), lambda t: (0, 0)),
        ],
        out_specs=[
            pl.BlockSpec((_RT, 2 * _C), lambda t: (t, 0)),
            pl.BlockSpec((8, 128), lambda t: (0, 0)),
        ],
        out_shape=[
            jax.ShapeDtypeStruct((_NROW, 2 * _C), jnp.float32),
            jax.ShapeDtypeStruct((8, 128), jnp.float32),
        ],
        scratch_shapes=[pltpu.VMEM((8, 128), jnp.float32)],
    )(y1, st1, g1r, be1r, w2t, b2r)


def _mlpC_body(z_ref, st_ref, g_ref, be_ref, o_ref):
    n = jnp.float32(_NROW)
    mean = st_ref[0:1, 0:2 * _C] / n
    var = st_ref[1:2, 0:2 * _C] / n - mean * mean
    xn = (z_ref[...] - mean) / jnp.sqrt(var + 1e-5) * g_ref[...] + be_ref[...]
    xr = jnp.maximum(xn, 0.0)
    o_ref[...] = jnp.max(xr.reshape(_RT // _K, _K, 2 * _C), axis=1)


def _run_mlpC(z, st2, g2r, be2r):
    grid = (_NROW // _RT,)
    return pl.pallas_call(
        _mlpC_body,
        grid=grid,
        in_specs=[
            pl.BlockSpec((_RT, 2 * _C), lambda t: (t, 0)),
            pl.BlockSpec((8, 128), lambda t: (0, 0)),
            pl.BlockSpec((1, 2 * _C), lambda t: (0, 0)),
            pl.BlockSpec((1, 2 * _C), lambda t: (0, 0)),
        ],
        out_specs=pl.BlockSpec((_RT // _K, 2 * _C), lambda t: (t, 0)),
        out_shape=jax.ShapeDtypeStruct((_B * _S, 2 * _C), jnp.float32),
    )(z, st2, g2r, be2r)


def kernel(xyz, feat, W1, b1, g1, be1, W2, b2, g2, be2):
    xyzt = xyz.transpose(0, 2, 1)
    far0 = jax.random.randint(jax.random.key(42), (_B,), 0, _N).astype(
        jnp.int32)
    far0_oh = (far0[:, None] == jnp.arange(_N, dtype=jnp.int32)[None, :]
               ).astype(jnp.float32)

    newxt = _run_fps(xyzt, far0_oh)
    new_xyz = newxt.transpose(0, 2, 1)

    gidx3 = _run_ball(xyzt, new_xyz)
    gidx = gidx3.reshape(_NROW)

    pts = jnp.concatenate(
        [xyz, feat, jnp.zeros((_B, _N, _CP - 3 - _C), jnp.float32)], axis=-1
    ).reshape(_B * _N, _CP)
    g = _run_sc_gather(pts, gidx)

    ctr = jnp.pad(
        jnp.repeat(new_xyz.reshape(_B * _S, 3), _K, axis=0),
        ((0, 0), (0, _CP - 3)))

    w1p = jnp.pad(W1.T, ((0, _CP - 3 - _C), (0, 0)))
    y1, st1 = _run_mlpA(g, ctr, w1p, b1.reshape(1, _C))
    z, st2 = _run_mlpB(y1, st1, g1.reshape(1, _C), be1.reshape(1, _C),
                       W2.T, b2.reshape(1, 2 * _C))
    out = _run_mlpC(z, st2, g2.reshape(1, 2 * _C), be2.reshape(1, 2 * _C))
    return (new_xyz, out.reshape(_B, _S, 2 * _C))


# speedup vs baseline: 1.2171x; 1.2171x over previous
import functools

import jax
import jax.numpy as jnp
from jax import lax
from jax.experimental import pallas as pl
from jax.experimental.pallas import tpu as pltpu
from jax.experimental.pallas import tpu_sc as plsc

_B = 4
_N = 8192
_C = 32
_S = 512
_K = 32
_R = 0.2
_CP = 48
_ST = 128
_RT = 4096
_NROW = _B * _S * _K


def _fps_body(xyzt_ref, far0_ref, newx_ref, dist_ref):
    x = xyzt_ref[:, 0, :]
    y = xyzt_ref[:, 1, :]
    z = xyzt_ref[:, 2, :]
    n_iota = lax.broadcasted_iota(jnp.int32, (_B, _N), 1)
    s_iota = lax.broadcasted_iota(jnp.int32, (_B, _S), 1)
    far0 = jnp.min(
        jnp.where(far0_ref[...] > 0.0, n_iota, jnp.int32(_N)), axis=1,
        keepdims=True)
    dist_ref[...] = jnp.full((_B, _N), 1e10, jnp.float32)

    def body(i, st):
        far, nx, ny, nz = st
        msk = n_iota == far
        cx = jnp.sum(jnp.where(msk, x, 0.0), axis=1, keepdims=True)
        cy = jnp.sum(jnp.where(msk, y, 0.0), axis=1, keepdims=True)
        cz = jnp.sum(jnp.where(msk, z, 0.0), axis=1, keepdims=True)
        s_msk = s_iota == i
        nx = jnp.where(s_msk, cx, nx)
        ny = jnp.where(s_msk, cy, ny)
        nz = jnp.where(s_msk, cz, nz)
        dx = x - cx
        dy = y - cy
        dz = z - cz
        d = dx * dx + dy * dy + dz * dz
        dist = jnp.minimum(dist_ref[...], d)
        dist_ref[...] = dist
        m = jnp.max(dist, axis=1, keepdims=True)
        far = jnp.min(jnp.where(dist == m, n_iota, jnp.int32(_N)), axis=1,
                      keepdims=True)
        return (far, nx, ny, nz)

    zero = jnp.zeros((_B, _S), jnp.float32)
    _, nx, ny, nz = lax.fori_loop(0, _S, body, (far0, zero, zero, zero))
    newx_ref[:, 0, :] = nx
    newx_ref[:, 1, :] = ny
    newx_ref[:, 2, :] = nz


def _run_fps(xyzt, far0_oh):
    return pl.pallas_call(
        _fps_body,
        out_shape=jax.ShapeDtypeStruct((_B, 3, _S), jnp.float32),
        scratch_shapes=[pltpu.VMEM((_B, _N), jnp.float32)],
    )(xyzt, far0_oh)


def _bf(v):
    return v.astype(jnp.bfloat16).astype(jnp.float32)


def _two_sum(p, q):
    s = p + q
    pp = s - q
    qq = s - pp
    return s, (p - pp) + (q - qq)


def _ball_body(xyzt_ref, nc_ref, idx_ref, dist_ref):
    b = pl.program_id(0)
    x = xyzt_ref[0, 0:1, :]
    y = xyzt_ref[0, 1:2, :]
    z = xyzt_ref[0, 2:3, :]
    c = nc_ref[0]
    l3 = lax.broadcasted_iota(jnp.int32, (_ST, 3), 1)
    cx = jnp.sum(jnp.where(l3 == 0, c, 0.0), axis=1, keepdims=True)
    cy = jnp.sum(jnp.where(l3 == 1, c, 0.0), axis=1, keepdims=True)
    cz = jnp.sum(jnp.where(l3 == 2, c, 0.0), axis=1, keepdims=True)
    a2 = (cx * cx + cy * cy) + cz * cz
    b2 = (x * x + y * y) + z * z
    p0 = _bf(cx) * _bf(x)
    p1 = _bf(cy) * _bf(y)
    p2 = _bf(cz) * _bf(z)
    s01, e01 = _two_sum(p0, p1)
    t, e2 = _two_sum(s01, p2)
    ab = t + (e01 + e2)
    dist_ref[...] = jnp.maximum((a2 + b2) - 2.0 * ab, 0.0)

    n_iota = lax.broadcasted_iota(jnp.int32, (_ST, _N), 1) + b * _N
    k_iota = lax.broadcasted_iota(jnp.int32, (_ST, _K), 1)
    big = jnp.int32(1 << 30)

    m0 = jnp.min(dist_ref[...], axis=1, keepdims=True)

    def rnd(k, st):
        out, first, m = st
        d = dist_ref[...]
        sel = jnp.min(jnp.where(d == m, n_iota, big), axis=1, keepdims=True)
        dnew = jnp.where(n_iota == sel, jnp.float32(3e38), d)
        dist_ref[...] = dnew
        mnext = jnp.min(dnew, axis=1, keepdims=True)
        first = jnp.where(k == 0, sel, first)
        chosen = jnp.where(jnp.sqrt(m) > _R, first, sel)
        out = jnp.where(k_iota == k, chosen, out)
        return (out, first, mnext)

    out0 = jnp.zeros((_ST, _K), jnp.int32)
    f0 = jnp.zeros((_ST, 1), jnp.int32)
    out, _, _ = lax.fori_loop(0, _K, rnd, (out0, f0, m0))
    idx_ref[0] = out


def _run_ball(xyzt, nc):
    grid = (_B, _S // _ST)
    return pl.pallas_call(
        _ball_body,
        grid=grid,
        in_specs=[
            pl.BlockSpec((1, 3, _N), lambda b, t: (b, 0, 0)),
            pl.BlockSpec((1, _ST, 3), lambda b, t: (b, t, 0)),
        ],
        out_specs=pl.BlockSpec((1, _ST, _K), lambda b, t: (b, t, 0)),
        out_shape=jax.ShapeDtypeStruct((_B, _S, _K), jnp.int32),
        scratch_shapes=[pltpu.VMEM((_ST, _N), jnp.float32)],
    )(xyzt, nc)


_NW = 32
_CH = 128
_NCHUNK = _NROW // (_NW * _CH)


def _sc_gather_body(pts_hbm, gidx_hbm, out_hbm, idx_v, rows_v, sem):
    wid = lax.axis_index("s") * 2 + lax.axis_index("c")

    def body(j, carry):
        base = (wid * _NCHUNK + j) * _CH
        pltpu.sync_copy(gidx_hbm.at[pl.ds(base, _CH)], idx_v)
        pltpu.async_copy(pts_hbm.at[idx_v], rows_v, sem).wait()
        pltpu.sync_copy(rows_v, out_hbm.at[pl.ds(base, _CH)])
        return carry

    lax.fori_loop(0, _NCHUNK, body, 0)


def _run_sc_gather(pts, gidx):
    mesh = plsc.VectorSubcoreMesh(core_axis_name="c", subcore_axis_name="s")
    fn = functools.partial(
        pl.kernel,
        mesh=mesh,
        compiler_params=pltpu.CompilerParams(use_tc_tiling_on_sc=False),
        out_type=jax.ShapeDtypeStruct((_NROW, _CP), jnp.float32),
        scratch_types=[
            pltpu.VMEM((_CH,), jnp.int32),
            pltpu.VMEM((_CH, _CP), jnp.float32),
            pltpu.SemaphoreType.DMA,
        ],
    )(_sc_gather_body)
    return fn(pts, gidx)


def _mlpA_body(g_ref, ctr_ref, w_ref, b_ref, y_ref, st_ref, acc_ref):
    t = pl.program_id(0)

    @pl.when(t == 0)
    def _():
        acc_ref[...] = jnp.zeros((8, 128), jnp.float32)

    xv = g_ref[...] - ctr_ref[...]
    y = lax.dot_general(xv, w_ref[...], (((1,), (0,)), ((), ())),
                        preferred_element_type=jnp.float32,
                        precision=lax.Precision.HIGHEST) + b_ref[...]
    y_ref[...] = y
    acc_ref[0:1, 0:_C] += jnp.sum(y, axis=0, keepdims=True)
    acc_ref[1:2, 0:_C] += jnp.sum(y * y, axis=0, keepdims=True)

    @pl.when(t == pl.num_programs(0) - 1)
    def _():
        st_ref[...] = acc_ref[...]


def _run_mlpA(g, ctr, w1p, b1r):
    grid = (_NROW // _RT,)
    return pl.pallas_call(
        _mlpA_body,
        grid=grid,
        in_specs=[
            pl.BlockSpec((_RT, _CP), lambda t: (t, 0)),
            pl.BlockSpec((_RT, _CP), lambda t: (t, 0)),
            pl.BlockSpec((_CP, _C), lambda t: (0, 0)),
            pl.BlockSpec((1, _C), lambda t: (0, 0)),
        ],
        out_specs=[
            pl.BlockSpec((_RT, _C), lambda t: (t, 0)),
            pl.BlockSpec((8, 128), lambda t: (0, 0)),
        ],
        out_shape=[
            jax.ShapeDtypeStruct((_NROW, _C), jnp.float32),
            jax.ShapeDtypeStruct((8, 128), jnp.float32),
        ],
        scratch_shapes=[pltpu.VMEM((8, 128), jnp.float32)],
    )(g, ctr, w1p, b1r)


def _mlpB_body(y_ref, st_ref, g_ref, be_ref, w_ref, b_ref, z_ref, st2_ref,
               acc_ref):
    t = pl.program_id(0)

    @pl.when(t == 0)
    def _():
        acc_ref[...] = jnp.zeros((8, 128), jnp.float32)

    n = jnp.float32(_NROW)
    mean = st_ref[0:1, 0:_C] / n
    var = st_ref[1:2, 0:_C] / n - mean * mean
    xn = (y_ref[...] - mean) / jnp.sqrt(var + 1e-5) * g_ref[...] + be_ref[...]
    xr = jnp.maximum(xn, 0.0)
    z = lax.dot_general(xr, w_ref[...], (((1,), (0,)), ((), ())),
                        preferred_element_type=jnp.float32,
                        precision=lax.Precision.HIGHEST) + b_ref[...]
    z_ref[...] = z
    acc_ref[0:1, 0:2 * _C] += jnp.sum(z, axis=0, keepdims=True)
    acc_ref[1:2, 0:2 * _C] += jnp.sum(z * z, axis=0, keepdims=True)

    @pl.when(t == pl.num_programs(0) - 1)
    def _():
        st2_ref[...] = acc_ref[...]


def _run_mlpB(y1, st1, g1r, be1r, w2t, b2r):
    grid = (_NROW // _RT,)
    return pl.pallas_call(
        _mlpB_body,
        grid=grid,
        in_specs=[
            pl.BlockSpec((_RT, _C), lambda t: (t, 0)),
            pl.BlockSpec((8, 128), lambda t: (0, 0)),
            pl.BlockSpec((1, _C), lambda t: (0, 0)),
            pl.BlockSpec((1, _C), lambda t: (0, 0)),
            pl.BlockSpec((_C, 2 * _C), lambda t: (0, 0)),
            pl.BlockSpec((1, 2 * _C), lambda t: (0, 0)),
        ],
        out_specs=[
            pl.BlockSpec((_RT, 2 * _C), lambda t: (t, 0)),
            pl.BlockSpec((8, 128), lambda t: (0, 0)),
        ],
        out_shape=[
            jax.ShapeDtypeStruct((_NROW, 2 * _C), jnp.float32),
            jax.ShapeDtypeStruct((8, 128), jnp.float32),
        ],
        scratch_shapes=[pltpu.VMEM((8, 128), jnp.float32)],
    )(y1, st1, g1r, be1r, w2t, b2r)


def _mlpC_body(z_ref, st_ref, g_ref, be_ref, o_ref):
    n = jnp.float32(_NROW)
    mean = st_ref[0:1, 0:2 * _C] / n
    var = st_ref[1:2, 0:2 * _C] / n - mean * mean
    xn = (z_ref[...] - mean) / jnp.sqrt(var + 1e-5) * g_ref[...] + be_ref[...]
    xr = jnp.maximum(xn, 0.0)
    o_ref[...] = jnp.max(xr.reshape(_RT // _K, _K, 2 * _C), axis=1)


def _run_mlpC(z, st2, g2r, be2r):
    grid = (_NROW // _RT,)
    return pl.pallas_call(
        _mlpC_body,
        grid=grid,
        in_specs=[
            pl.BlockSpec((_RT, 2 * _C), lambda t: (t, 0)),
            pl.BlockSpec((8, 128), lambda t: (0, 0)),
            pl.BlockSpec((1, 2 * _C), lambda t: (0, 0)),
            pl.BlockSpec((1, 2 * _C), lambda t: (0, 0)),
        ],
        out_specs=pl.BlockSpec((_RT // _K, 2 * _C), lambda t: (t, 0)),
        out_shape=jax.ShapeDtypeStruct((_B * _S, 2 * _C), jnp.float32),
    )(z, st2, g2r, be2r)


def kernel(xyz, feat, W1, b1, g1, be1, W2, b2, g2, be2):
    xyzt = xyz.transpose(0, 2, 1)
    far0 = jax.random.randint(jax.random.key(42), (_B,), 0, _N).astype(
        jnp.int32)
    far0_oh = (far0[:, None] == jnp.arange(_N, dtype=jnp.int32)[None, :]
               ).astype(jnp.float32)

    newxt = _run_fps(xyzt, far0_oh)
    new_xyz = newxt.transpose(0, 2, 1)

    gidx3 = _run_ball(xyzt, new_xyz)
    gidx = gidx3.reshape(_NROW)

    pts = jnp.concatenate(
        [xyz, feat, jnp.zeros((_B, _N, _CP - 3 - _C), jnp.float32)], axis=-1
    ).reshape(_B * _N, _CP)
    g = _run_sc_gather(pts, gidx)

    ctr = jnp.pad(
        jnp.repeat(new_xyz.reshape(_B * _S, 3), _K, axis=0),
        ((0, 0), (0, _CP - 3)))

    w1p = jnp.pad(W1.T, ((0, _CP - 3 - _C), (0, 0)))
    y1, st1 = _run_mlpA(g, ctr, w1p, b1.reshape(1, _C))
    z, st2 = _run_mlpB(y1, st1, g1.reshape(1, _C), be1.reshape(1, _C),
                       W2.T, b2.reshape(1, 2 * _C))
    out = _run_mlpC(z, st2, g2.reshape(1, 2 * _C), be2.reshape(1, 2 * _C))
    return (new_xyz, out.reshape(_B, _S, 2 * _C))


# speedup vs baseline: 1.2728x; 1.0458x over previous
import functools

import jax
import jax.numpy as jnp
from jax import lax
from jax.experimental import pallas as pl
from jax.experimental.pallas import tpu as pltpu
from jax.experimental.pallas import tpu_sc as plsc

_B = 4
_N = 8192
_C = 32
_S = 512
_K = 32
_R = 0.2
_CP = 48
_ST = 128
_RT = 4096
_NROW = _B * _S * _K


def _fps_body(xyzt_ref, far0_ref, newx_ref, dist_ref):
    x = xyzt_ref[:, 0, :]
    y = xyzt_ref[:, 1, :]
    z = xyzt_ref[:, 2, :]
    n_iota = lax.broadcasted_iota(jnp.int32, (_B, _N), 1)
    s_iota = lax.broadcasted_iota(jnp.int32, (_B, _S), 1)
    far0 = jnp.min(
        jnp.where(far0_ref[...] > 0.0, n_iota, jnp.int32(_N)), axis=1,
        keepdims=True)
    dist_ref[...] = jnp.full((_B, _N), 1e10, jnp.float32)

    def body(i, st):
        far, nx, ny, nz = st
        msk = n_iota == far
        cx = jnp.sum(jnp.where(msk, x, 0.0), axis=1, keepdims=True)
        cy = jnp.sum(jnp.where(msk, y, 0.0), axis=1, keepdims=True)
        cz = jnp.sum(jnp.where(msk, z, 0.0), axis=1, keepdims=True)
        s_msk = s_iota == i
        nx = jnp.where(s_msk, cx, nx)
        ny = jnp.where(s_msk, cy, ny)
        nz = jnp.where(s_msk, cz, nz)
        dx = x - cx
        dy = y - cy
        dz = z - cz
        d = dx * dx + dy * dy + dz * dz
        dist = jnp.minimum(dist_ref[...], d)
        dist_ref[...] = dist
        m = jnp.max(dist, axis=1, keepdims=True)
        far = jnp.min(jnp.where(dist == m, n_iota, jnp.int32(_N)), axis=1,
                      keepdims=True)
        return (far, nx, ny, nz)

    zero = jnp.zeros((_B, _S), jnp.float32)
    _, nx, ny, nz = lax.fori_loop(0, _S, body, (far0, zero, zero, zero))
    newx_ref[:, 0, :] = nx
    newx_ref[:, 1, :] = ny
    newx_ref[:, 2, :] = nz


def _run_fps(xyzt, far0_oh):
    return pl.pallas_call(
        _fps_body,
        out_shape=jax.ShapeDtypeStruct((_B, 3, _S), jnp.float32),
        scratch_shapes=[pltpu.VMEM((_B, _N), jnp.float32)],
    )(xyzt, far0_oh)


def _bf(v):
    return v.astype(jnp.bfloat16).astype(jnp.float32)


def _two_sum(p, q):
    s = p + q
    pp = s - q
    qq = s - pp
    return s, (p - pp) + (q - qq)


def _ball_body(xyzt_ref, nc_ref, idx_ref, dist_ref):
    b = pl.program_id(0)
    x = xyzt_ref[0, 0:1, :]
    y = xyzt_ref[0, 1:2, :]
    z = xyzt_ref[0, 2:3, :]
    c = nc_ref[0]
    l3 = lax.broadcasted_iota(jnp.int32, (_ST, 3), 1)
    cx = jnp.sum(jnp.where(l3 == 0, c, 0.0), axis=1, keepdims=True)
    cy = jnp.sum(jnp.where(l3 == 1, c, 0.0), axis=1, keepdims=True)
    cz = jnp.sum(jnp.where(l3 == 2, c, 0.0), axis=1, keepdims=True)
    a2 = (cx * cx + cy * cy) + cz * cz
    b2 = (x * x + y * y) + z * z
    p0 = _bf(cx) * _bf(x)
    p1 = _bf(cy) * _bf(y)
    p2 = _bf(cz) * _bf(z)
    s01, e01 = _two_sum(p0, p1)
    t, e2 = _two_sum(s01, p2)
    ab = t + (e01 + e2)
    dist_ref[...] = jnp.maximum((a2 + b2) - 2.0 * ab, 0.0)

    n_iota = lax.broadcasted_iota(jnp.int32, (_ST, _N), 1) + b * _N
    k_iota = lax.broadcasted_iota(jnp.int32, (_ST, _K), 1)
    big = jnp.int32(1 << 30)

    def rnd(k, st):
        out, first = st
        d = dist_ref[...]
        m = jnp.min(d, axis=1, keepdims=True)
        sel = jnp.min(jnp.where(d == m, n_iota, big), axis=1, keepdims=True)
        dist_ref[...] = jnp.where(n_iota == sel, jnp.float32(3e38), d)
        first = jnp.where(k == 0, sel, first)
        chosen = jnp.where(jnp.sqrt(m) > _R, first, sel)
        out = jnp.where(k_iota == k, chosen, out)
        return (out, first)

    out0 = jnp.zeros((_ST, _K), jnp.int32)
    f0 = jnp.zeros((_ST, 1), jnp.int32)
    out, _ = lax.fori_loop(0, _K, rnd, (out0, f0))
    idx_ref[0] = out


def _run_ball(xyzt, nc):
    grid = (_B, _S // _ST)
    return pl.pallas_call(
        _ball_body,
        grid=grid,
        in_specs=[
            pl.BlockSpec((1, 3, _N), lambda b, t: (b, 0, 0)),
            pl.BlockSpec((1, _ST, 3), lambda b, t: (b, t, 0)),
        ],
        out_specs=pl.BlockSpec((1, _ST, _K), lambda b, t: (b, t, 0)),
        out_shape=jax.ShapeDtypeStruct((_B, _S, _K), jnp.int32),
        scratch_shapes=[pltpu.VMEM((_ST, _N), jnp.float32)],
    )(xyzt, nc)


_NW = 32
_CH = 128
_NCHUNK = _NROW // (_NW * _CH)


def _sc_gather_body(pts_hbm, gidx_hbm, out_hbm, idx_v, rows_v, sem):
    wid = lax.axis_index("s") * 2 + lax.axis_index("c")

    def body(j, carry):
        base = (wid * _NCHUNK + j) * _CH
        pltpu.sync_copy(gidx_hbm.at[pl.ds(base, _CH)], idx_v)
        pltpu.async_copy(pts_hbm.at[idx_v], rows_v, sem).wait()
        pltpu.sync_copy(rows_v, out_hbm.at[pl.ds(base, _CH)])
        return carry

    lax.fori_loop(0, _NCHUNK, body, 0)


def _run_sc_gather(pts, gidx):
    mesh = plsc.VectorSubcoreMesh(core_axis_name="c", subcore_axis_name="s")
    fn = functools.partial(
        pl.kernel,
        mesh=mesh,
        compiler_params=pltpu.CompilerParams(use_tc_tiling_on_sc=False),
        out_type=jax.ShapeDtypeStruct((_NROW, _CP), jnp.float32),
        scratch_types=[
            pltpu.VMEM((_CH,), jnp.int32),
            pltpu.VMEM((_CH, _CP), jnp.float32),
            pltpu.SemaphoreType.DMA,
        ],
    )(_sc_gather_body)
    return fn(pts, gidx)


def _mlpA_body(g_ref, ctr_ref, w_ref, b_ref, y_ref, st_ref, acc_ref):
    t = pl.program_id(0)

    @pl.when(t == 0)
    def _():
        acc_ref[...] = jnp.zeros((8, 128), jnp.float32)

    xv = g_ref[...] - ctr_ref[...]
    y = lax.dot_general(xv, w_ref[...], (((1,), (0,)), ((), ())),
                        preferred_element_type=jnp.float32,
                        precision=lax.Precision.HIGHEST) + b_ref[...]
    y_ref[...] = y
    acc_ref[0:1, 0:_C] += jnp.sum(y, axis=0, keepdims=True)
    acc_ref[1:2, 0:_C] += jnp.sum(y * y, axis=0, keepdims=True)

    @pl.when(t == pl.num_programs(0) - 1)
    def _():
        st_ref[...] = acc_ref[...]


def _run_mlpA(g, ctr, w1p, b1r):
    grid = (_NROW // _RT,)
    return pl.pallas_call(
        _mlpA_body,
        grid=grid,
        in_specs=[
            pl.BlockSpec((_RT, _CP), lambda t: (t, 0)),
            pl.BlockSpec((_RT, _CP), lambda t: (t, 0)),
            pl.BlockSpec((_CP, _C), lambda t: (0, 0)),
            pl.BlockSpec((1, _C), lambda t: (0, 0)),
        ],
        out_specs=[
            pl.BlockSpec((_RT, _C), lambda t: (t, 0)),
            pl.BlockSpec((8, 128), lambda t: (0, 0)),
        ],
        out_shape=[
            jax.ShapeDtypeStruct((_NROW, _C), jnp.float32),
            jax.ShapeDtypeStruct((8, 128), jnp.float32),
        ],
        scratch_shapes=[pltpu.VMEM((8, 128), jnp.float32)],
    )(g, ctr, w1p, b1r)


def _mlpB_body(y_ref, st_ref, g_ref, be_ref, w_ref, b_ref, z_ref, st2_ref,
               acc_ref):
    t = pl.program_id(0)

    @pl.when(t == 0)
    def _():
        acc_ref[...] = jnp.zeros((8, 128), jnp.float32)

    n = jnp.float32(_NROW)
    mean = st_ref[0:1, 0:_C] / n
    var = st_ref[1:2, 0:_C] / n - mean * mean
    xn = (y_ref[...] - mean) / jnp.sqrt(var + 1e-5) * g_ref[...] + be_ref[...]
    xr = jnp.maximum(xn, 0.0)
    z = lax.dot_general(xr, w_ref[...], (((1,), (0,)), ((), ())),
                        preferred_element_type=jnp.float32,
                        precision=lax.Precision.HIGHEST) + b_ref[...]
    z_ref[...] = z
    acc_ref[0:1, 0:2 * _C] += jnp.sum(z, axis=0, keepdims=True)
    acc_ref[1:2, 0:2 * _C] += jnp.sum(z * z, axis=0, keepdims=True)

    @pl.when(t == pl.num_programs(0) - 1)
    def _():
        st2_ref[...] = acc_ref[...]


def _run_mlpB(y1, st1, g1r, be1r, w2t, b2r):
    grid = (_NROW // _RT,)
    return pl.pallas_call(
        _mlpB_body,
        grid=grid,
        in_specs=[
            pl.BlockSpec((_RT, _C), lambda t: (t, 0)),
            pl.BlockSpec((8, 128), lambda t: (0, 0)),
            pl.BlockSpec((1, _C), lambda t: (0, 0)),
            pl.BlockSpec((1, _C), lambda t: (0, 0)),
            pl.BlockSpec((_C, 2 * _C), lambda t: (0, 0)),
            pl.BlockSpec((1, 2 * _C), lambda t: (0, 0)),
        ],
        out_specs=[
            pl.BlockSpec((_RT, 2 * _C), lambda t: (t, 0)),
            pl.BlockSpec((8, 128), lambda t: (0, 0)),
        ],
        out_shape=[
            jax.ShapeDtypeStruct((_NROW, 2 * _C), jnp.float32),
            jax.ShapeDtypeStruct((8, 128), jnp.float32),
        ],
        scratch_shapes=[pltpu.VMEM((8, 128), jnp.float32)],
    )(y1, st1, g1r, be1r, w2t, b2r)


def _mlpC_body(z_ref, st_ref, g_ref, be_ref, o_ref):
    n = jnp.float32(_NROW)
    mean = st_ref[0:1, 0:2 * _C] / n
    var = st_ref[1:2, 0:2 * _C] / n - mean * mean
    xn = (z_ref[...] - mean) / jnp.sqrt(var + 1e-5) * g_ref[...] + be_ref[...]
    xr = jnp.maximum(xn, 0.0)
    o_ref[...] = jnp.max(xr.reshape(_RT // _K, _K, 2 * _C), axis=1)


def _run_mlpC(z, st2, g2r, be2r):
    grid = (_NROW // _RT,)
    return pl.pallas_call(
        _mlpC_body,
        grid=grid,
        in_specs=[
            pl.BlockSpec((_RT, 2 * _C), lambda t: (t, 0)),
            pl.BlockSpec((8, 128), lambda t: (0, 0)),
            pl.BlockSpec((1, 2 * _C), lambda t: (0, 0)),
            pl.BlockSpec((1, 2 * _C), lambda t: (0, 0)),
        ],
        out_specs=pl.BlockSpec((_RT // _K, 2 * _C), lambda t: (t, 0)),
        out_shape=jax.ShapeDtypeStruct((_B * _S, 2 * _C), jnp.float32),
    )(z, st2, g2r, be2r)


def kernel(xyz, feat, W1, b1, g1, be1, W2, b2, g2, be2):
    xyzt = xyz.transpose(0, 2, 1)
    far0 = jax.random.randint(jax.random.key(42), (_B,), 0, _N).astype(
        jnp.int32)
    far0_oh = (far0[:, None] == jnp.arange(_N, dtype=jnp.int32)[None, :]
               ).astype(jnp.float32)

    newxt = _run_fps(xyzt, far0_oh)
    new_xyz = newxt.transpose(0, 2, 1)

    gidx3 = _run_ball(xyzt, new_xyz)
    gidx = gidx3.reshape(_NROW)

    pts = jnp.concatenate(
        [xyz, feat, jnp.zeros((_B, _N, _CP - 3 - _C), jnp.float32)], axis=-1
    ).reshape(_B * _N, _CP)
    g = _run_sc_gather(pts, gidx)

    ctr = jnp.pad(
        jnp.repeat(new_xyz.reshape(_B * _S, 3), _K, axis=0),
        ((0, 0), (0, _CP - 3)))

    w1p = jnp.pad(W1.T, ((0, _CP - 3 - _C), (0, 0)))
    y1, st1 = _run_mlpA(g, ctr, w1p, b1.reshape(1, _C))
    z, st2 = _run_mlpB(y1, st1, g1.reshape(1, _C), be1.reshape(1, _C),
                       W2.T, b2.reshape(1, 2 * _C))
    out = _run_mlpC(z, st2, g2.reshape(1, 2 * _C), be2.reshape(1, 2 * _C))
    return (new_xyz, out.reshape(_B, _S, 2 * _C))
